# Initial kernel scaffold; baseline (speedup 1.0000x reference)
import jax
import jax.numpy as jnp
from jax.experimental import pallas as pl


def kernel(board, emb_table, W1, b1, W2, b2, W3, b3):
    raise NotImplementedError("write your pallas kernel here")


# speedup vs baseline: 4.3303x; 4.3303x over previous
import jax
import jax.numpy as jnp
from jax import lax
from jax.experimental import pallas as pl
from jax.experimental.pallas import tpu as pltpu
from jax.experimental.pallas import tpu_sc as plsc

_N_TILES = 1024
_EMBED = 32
_HID = 256
_NA = 4
_B = 4096

_NC, _NS = 2, 16
_NW = _NC * _NS
_BPW = _B // _NW
_CHUNK = 128
_NCHUNK = _N_TILES // _CHUNK


def _sc_gather_body(table_hbm, board_hbm, out_hbm, idx_v, emb_v, sem):
    w = lax.axis_index("s") * _NC + lax.axis_index("c")
    base = w * _BPW

    def row_body(i, carry):
        br = base + i
        pltpu.sync_copy(board_hbm.at[br], idx_v)
        copies = []
        for j in range(_NCHUNK):
            copies.append(pltpu.async_copy(
                table_hbm.at[idx_v.at[j]],
                emb_v.at[pl.ds(j * _CHUNK, _CHUNK), :],
                sem,
            ))
        for c in copies:
            c.wait()
        pltpu.sync_copy(emb_v, out_hbm.at[br])
        return carry

    lax.fori_loop(0, _BPW, row_body, 0)


def _sc_gather(emb_table, board3):
    mesh = plsc.VectorSubcoreMesh(core_axis_name="c", subcore_axis_name="s")
    f = pl.kernel(
        _sc_gather_body,
        out_type=jax.ShapeDtypeStruct((_B, _N_TILES, _EMBED), jnp.float32),
        mesh=mesh,
        scratch_types=[
            pltpu.VMEM((_NCHUNK, _CHUNK), jnp.int32),
            pltpu.VMEM((_N_TILES, _EMBED), jnp.float32),
            pltpu.SemaphoreType.DMA,
        ],
        compiler_params=pltpu.CompilerParams(use_tc_tiling_on_sc=False),
    )
    return f(emb_table, board3)


_BB = 512
_KB = 4096
_K = _N_TILES * _EMBED


def _mlp_body(flat_ref, w1_ref, b1_ref, w2_ref, b2_ref, w3_ref, b3_ref,
              out_ref, acc_ref):
    k = pl.program_id(0)
    b = pl.program_id(1)
    nk = pl.num_programs(0)
    part = jnp.dot(flat_ref[...], w1_ref[...], preferred_element_type=jnp.float32)
    sl = pl.ds(b * _BB, _BB)

    @pl.when(k == 0)
    def _():
        acc_ref[sl, :] = part

    @pl.when(k > 0)
    def _():
        acc_ref[sl, :] = acc_ref[sl, :] + part

    @pl.when(k == nk - 1)
    def _():
        h1 = jnp.maximum(acc_ref[sl, :] + b1_ref[...], 0.0)
        h2 = jnp.dot(h1, w2_ref[...], preferred_element_type=jnp.float32)
        h2 = jnp.maximum(h2 + b2_ref[...], 0.0)
        out_ref[...] = (
            jnp.dot(h2, w3_ref[...], preferred_element_type=jnp.float32)
            + b3_ref[...]
        )


def _tc_mlp(flat, W1, b1, W2, b2, W3, b3):
    grid = (_K // _KB, _B // _BB)
    return pl.pallas_call(
        _mlp_body,
        grid=grid,
        in_specs=[
            pl.BlockSpec((_BB, _KB), lambda k, b: (b, k)),
            pl.BlockSpec((_KB, _HID), lambda k, b: (k, 0)),
            pl.BlockSpec((1, _HID), lambda k, b: (0, 0)),
            pl.BlockSpec((_HID, _HID), lambda k, b: (0, 0)),
            pl.BlockSpec((1, _HID), lambda k, b: (0, 0)),
            pl.BlockSpec((_HID, _NA), lambda k, b: (0, 0)),
            pl.BlockSpec((1, _NA), lambda k, b: (0, 0)),
        ],
        out_specs=pl.BlockSpec((_BB, _NA), lambda k, b: (b, 0)),
        out_shape=jax.ShapeDtypeStruct((_B, _NA), jnp.float32),
        scratch_shapes=[pltpu.VMEM((_B, _HID), jnp.float32)],
        compiler_params=pltpu.CompilerParams(
            dimension_semantics=("arbitrary", "arbitrary"),
        ),
    )(flat, W1, b1, W2, b2, W3, b3)


def kernel(board, emb_table, W1, b1, W2, b2, W3, b3):
    board3 = board.astype(jnp.int32).reshape(_B, _NCHUNK, _CHUNK)
    emb3d = _sc_gather(emb_table, board3)
    flat = emb3d.reshape(_B, _K)
    return _tc_mlp(
        flat, W1, b1.reshape(1, _HID), W2, b2.reshape(1, _HID),
        W3, b3.reshape(1, _NA),
    )
